# Initial kernel scaffold; baseline (speedup 1.0000x reference)
import jax
import jax.numpy as jnp
from jax.experimental import pallas as pl


def kernel(x, edge_index, ptr, W1, a_src1, a_dst1, b1, W2, a_src2, a_dst2, b2, Ws1, bs1, Ws2, bs2, Ws3, bs3, Wm1, bm1, Wm2, bm2, Wm3, bm3):
    raise NotImplementedError("write your pallas kernel here")


# speedup vs baseline: 1.0006x; 1.0006x over previous
import functools

import jax
import jax.numpy as jnp
from jax.experimental import pallas as pl
from jax.experimental.pallas import tpu as pltpu

N = 10000
E = 320000
D = 128
HEADS = 8
OUT = 128
HO = HEADS * OUT
G = 8
GS = N // G

ROW_BLK = 400


def _mm_e_body(x_ref, w_ref, ae_ref, h_ref, eo_ref):
    h = jnp.dot(x_ref[...], w_ref[...], preferred_element_type=jnp.float32)
    h_ref[...] = h
    eo_ref[...] = jnp.dot(h, ae_ref[...], preferred_element_type=jnp.float32)


def _mm_e(x, W, AE):
    din = x.shape[1]
    grid = (N // ROW_BLK,)
    return pl.pallas_call(
        _mm_e_body,
        grid=grid,
        in_specs=[
            pl.BlockSpec((ROW_BLK, din), lambda i: (i, 0)),
            pl.BlockSpec((din, HO), lambda i: (0, 0)),
            pl.BlockSpec((HO, 2 * HEADS), lambda i: (0, 0)),
        ],
        out_specs=[
            pl.BlockSpec((ROW_BLK, HO), lambda i: (i, 0)),
            pl.BlockSpec((ROW_BLK, 2 * HEADS), lambda i: (i, 0)),
        ],
        out_shape=[
            jax.ShapeDtypeStruct((N, HO), jnp.float32),
            jax.ShapeDtypeStruct((N, 2 * HEADS), jnp.float32),
        ],
    )(x, W, AE)


def _edge_phase(h, es, ed, src, dst):
    e = es[src] + ed[dst]
    e = jnp.where(e > 0, e, 0.2 * e)
    m = jax.ops.segment_max(e, dst, num_segments=N)
    p = jnp.exp(e - m[dst])
    s = jax.ops.segment_sum(p, dst, num_segments=N)
    alpha = p / (s[dst] + 1e-16)
    hh = h.reshape(N, HEADS, OUT)
    agg = jax.ops.segment_sum(hh[src] * alpha[:, :, None], dst, num_segments=N)
    return agg.reshape(N, HO)


def _tail_body(h_ref, ws1_ref, bs1_ref, ws2_ref, bs2_ref, ws3_ref, bs3_ref, hh_ref):
    hb = h_ref[0]
    nrm = jnp.sqrt(jnp.sum(hb * hb, axis=1, keepdims=True))
    hn = hb / jnp.clip(nrm, 1e-12, None)
    g = jnp.maximum(jnp.dot(hn, ws1_ref[...], preferred_element_type=jnp.float32) + bs1_ref[...], 0.0)
    g = jnp.maximum(jnp.dot(g, ws2_ref[...], preferred_element_type=jnp.float32) + bs2_ref[...], 0.0)
    g = jnp.maximum(jnp.dot(g, ws3_ref[...], preferred_element_type=jnp.float32) + bs3_ref[...], 0.0)
    hh = jax.lax.dot_general(g, g, (((0,), (0,)), ((), ())),
                             preferred_element_type=jnp.float32)
    hh_ref[0] = hh


def _tail(h2, Ws1, bs1, Ws2, bs2, Ws3, bs3):
    hg = h2.reshape(G, GS, HO)
    return pl.pallas_call(
        _tail_body,
        grid=(G,),
        in_specs=[
            pl.BlockSpec((1, GS, HO), lambda i: (i, 0, 0)),
            pl.BlockSpec((HO, 32), lambda i: (0, 0)),
            pl.BlockSpec((1, 32), lambda i: (0, 0)),
            pl.BlockSpec((32, 32), lambda i: (0, 0)),
            pl.BlockSpec((1, 32), lambda i: (0, 0)),
            pl.BlockSpec((32, 32), lambda i: (0, 0)),
            pl.BlockSpec((1, 32), lambda i: (0, 0)),
        ],
        out_specs=pl.BlockSpec((1, 32, 32), lambda i: (i, 0, 0)),
        out_shape=jax.ShapeDtypeStruct((G, 32, 32), jnp.float32),
    )(hg, Ws1, bs1.reshape(1, 32), Ws2, bs2.reshape(1, 32), Ws3, bs3.reshape(1, 32))


def _head_body(hh_ref, wm1_ref, bm1_ref, wm2_ref, bm2_ref, wm3_ref, bm3_ref, o_ref):
    o = jnp.maximum(jnp.dot(hh_ref[...], wm1_ref[...], preferred_element_type=jnp.float32) + bm1_ref[...], 0.0)
    o = jnp.maximum(jnp.dot(o, wm2_ref[...], preferred_element_type=jnp.float32) + bm2_ref[...], 0.0)
    o = jnp.maximum(jnp.dot(o, wm3_ref[...], preferred_element_type=jnp.float32) + bm3_ref[...], 0.0)
    o_ref[...] = o


def _head(HH, Wm1, bm1, Wm2, bm2, Wm3, bm3):
    return pl.pallas_call(
        _head_body,
        out_shape=jax.ShapeDtypeStruct((G, 2), jnp.float32),
    )(HH, Wm1, bm1.reshape(1, 32), Wm2, bm2.reshape(1, 32), Wm3, bm3.reshape(1, 2))


def _block_diag_ae(a_src, a_dst):
    eye = jnp.eye(HEADS, dtype=jnp.float32)
    asm = (eye[:, None, :] * a_src[:, :, None]).reshape(HO, HEADS)
    adm = (eye[:, None, :] * a_dst[:, :, None]).reshape(HO, HEADS)
    return jnp.concatenate([asm, adm], axis=1)


def kernel(x, edge_index, ptr, W1, a_src1, a_dst1, b1, W2, a_src2, a_dst2, b2,
           Ws1, bs1, Ws2, bs2, Ws3, bs3, Wm1, bm1, Wm2, bm2, Wm3, bm3):
    loop = jnp.arange(N, dtype=edge_index.dtype)
    src = jnp.concatenate([edge_index[0], loop])
    dst = jnp.concatenate([edge_index[1], loop])

    AE1 = _block_diag_ae(a_src1, a_dst1)
    AE2 = _block_diag_ae(a_src2, a_dst2)

    h, eo = _mm_e(x, W1, AE1)
    agg = _edge_phase(h, eo[:, :HEADS], eo[:, HEADS:], src, dst)
    h1 = jnp.maximum(agg + b1[None, :], 0.0)

    h, eo = _mm_e(h1, W2, AE2)
    agg = _edge_phase(h, eo[:, :HEADS], eo[:, HEADS:], src, dst)
    h2 = jnp.maximum(agg + b2[None, :], 0.0)

    HHm = _tail(h2, Ws1, bs1, Ws2, bs2, Ws3, bs3)
    HH = HHm.reshape(G, HO)
    o = _head(HH, Wm1, bm1, Wm2, bm2, Wm3, bm3)
    return (HH, o)


# speedup vs baseline: 1.0006x; 1.0000x over previous
import jax
import jax.numpy as jnp
from jax.experimental import pallas as pl

N = 10000
E = 320000
D = 128
HEADS = 8
OUT = 128
HO = HEADS * OUT
G = 8
GS = N // G

ROW_BLK = 400


def _mm_e_body(x_ref, w_ref, ae_ref, h_ref, eo_ref):
    h = jnp.dot(x_ref[...], w_ref[...], preferred_element_type=jnp.float32)
    h_ref[...] = h
    eo_ref[...] = jnp.dot(h, ae_ref[...], preferred_element_type=jnp.float32)


def _mm_e(x, W, AE):
    din = x.shape[1]
    return pl.pallas_call(
        _mm_e_body,
        grid=(N // ROW_BLK,),
        in_specs=[
            pl.BlockSpec((ROW_BLK, din), lambda i: (i, 0)),
            pl.BlockSpec((din, HO), lambda i: (0, 0)),
            pl.BlockSpec((HO, 2 * HEADS), lambda i: (0, 0)),
        ],
        out_specs=[
            pl.BlockSpec((ROW_BLK, HO), lambda i: (i, 0)),
            pl.BlockSpec((ROW_BLK, 2 * HEADS), lambda i: (i, 0)),
        ],
        out_shape=[
            jax.ShapeDtypeStruct((N, HO), jnp.float32),
            jax.ShapeDtypeStruct((N, 2 * HEADS), jnp.float32),
        ],
    )(x, W, AE)


def _edge_phase(h, es, ed, src, dst):
    e = es[src] + ed[dst]
    e = jnp.where(e > 0, e, 0.2 * e)
    m = jax.ops.segment_max(e, dst, num_segments=N)
    p = jnp.exp(e - m[dst])
    s = jax.ops.segment_sum(p, dst, num_segments=N)
    alpha = p / (s[dst] + 1e-16)
    hh = h.reshape(N, HEADS, OUT)
    agg = jax.ops.segment_sum(hh[src] * alpha[:, :, None], dst, num_segments=N)
    return agg.reshape(N, HO)


def _tail_body(h_ref, ws1_ref, bs1_ref, ws2_ref, bs2_ref, ws3_ref, bs3_ref, hh_ref):
    hb = h_ref[0]
    nrm = jnp.sqrt(jnp.sum(hb * hb, axis=1, keepdims=True))
    hn = hb / jnp.clip(nrm, 1e-12, None)
    g = jnp.maximum(jnp.dot(hn, ws1_ref[...], preferred_element_type=jnp.float32) + bs1_ref[...], 0.0)
    g = jnp.maximum(jnp.dot(g, ws2_ref[...], preferred_element_type=jnp.float32) + bs2_ref[...], 0.0)
    g = jnp.maximum(jnp.dot(g, ws3_ref[...], preferred_element_type=jnp.float32) + bs3_ref[...], 0.0)
    hh = jax.lax.dot_general(g, g, (((0,), (0,)), ((), ())),
                             preferred_element_type=jnp.float32)
    hh_ref[0] = hh


def _tail(h2, Ws1, bs1, Ws2, bs2, Ws3, bs3):
    hg = h2.reshape(G, GS, HO)
    return pl.pallas_call(
        _tail_body,
        grid=(G,),
        in_specs=[
            pl.BlockSpec((1, GS, HO), lambda i: (i, 0, 0)),
            pl.BlockSpec((HO, 32), lambda i: (0, 0)),
            pl.BlockSpec((1, 32), lambda i: (0, 0)),
            pl.BlockSpec((32, 32), lambda i: (0, 0)),
            pl.BlockSpec((1, 32), lambda i: (0, 0)),
            pl.BlockSpec((32, 32), lambda i: (0, 0)),
            pl.BlockSpec((1, 32), lambda i: (0, 0)),
        ],
        out_specs=pl.BlockSpec((1, 32, 32), lambda i: (i, 0, 0)),
        out_shape=jax.ShapeDtypeStruct((G, 32, 32), jnp.float32),
    )(hg, Ws1, bs1.reshape(1, 32), Ws2, bs2.reshape(1, 32), Ws3, bs3.reshape(1, 32))


def _head_body(hh_ref, wm1_ref, bm1_ref, wm2_ref, bm2_ref, wm3_ref, bm3_ref, o_ref):
    o = jnp.maximum(jnp.dot(hh_ref[...], wm1_ref[...], preferred_element_type=jnp.float32) + bm1_ref[...], 0.0)
    o = jnp.maximum(jnp.dot(o, wm2_ref[...], preferred_element_type=jnp.float32) + bm2_ref[...], 0.0)
    o = jnp.maximum(jnp.dot(o, wm3_ref[...], preferred_element_type=jnp.float32) + bm3_ref[...], 0.0)
    o_ref[...] = o


def _head(HH, Wm1, bm1, Wm2, bm2, Wm3, bm3):
    return pl.pallas_call(
        _head_body,
        out_shape=jax.ShapeDtypeStruct((G, 2), jnp.float32),
    )(HH, Wm1, bm1.reshape(1, 32), Wm2, bm2.reshape(1, 32), Wm3, bm3.reshape(1, 2))


def _block_diag_ae(a_src, a_dst):
    eye = jnp.eye(HEADS, dtype=jnp.float32)
    asm = (eye[:, None, :] * a_src[:, :, None]).reshape(HO, HEADS)
    adm = (eye[:, None, :] * a_dst[:, :, None]).reshape(HO, HEADS)
    return jnp.concatenate([asm, adm], axis=1)


def kernel(x, edge_index, ptr, W1, a_src1, a_dst1, b1, W2, a_src2, a_dst2, b2,
           Ws1, bs1, Ws2, bs2, Ws3, bs3, Wm1, bm1, Wm2, bm2, Wm3, bm3):
    loop = jnp.arange(N, dtype=edge_index.dtype)
    src = jnp.concatenate([edge_index[0], loop])
    dst = jnp.concatenate([edge_index[1], loop])

    AE1 = _block_diag_ae(a_src1, a_dst1)
    AE2 = _block_diag_ae(a_src2, a_dst2)

    h, eo = _mm_e(x, W1, AE1)
    agg = _edge_phase(h, eo[:, :HEADS], eo[:, HEADS:], src, dst)
    h1 = jnp.maximum(agg + b1[None, :], 0.0)

    h, eo = _mm_e(h1, W2, AE2)
    agg = _edge_phase(h, eo[:, :HEADS], eo[:, HEADS:], src, dst)
    h2 = jnp.maximum(agg + b2[None, :], 0.0)

    HHm = _tail(h2, Ws1, bs1, Ws2, bs2, Ws3, bs3)
    HH = HHm.reshape(G, HO)
    o = _head(HH, Wm1, bm1, Wm2, bm2, Wm3, bm3)
    return (HH, o)
